# Initial kernel scaffold; baseline (speedup 1.0000x reference)
import jax
import jax.numpy as jnp
from jax.experimental import pallas as pl


def kernel(values, tok_table, pos_table):
    raise NotImplementedError("write your pallas kernel here")


# speedup vs baseline: 1.0417x; 1.0417x over previous
import functools

import jax
import jax.numpy as jnp
from jax import lax
from jax.experimental import pallas as pl
from jax.experimental.pallas import tpu as pltpu
from jax.experimental.pallas import tpu_sc as plsc

VOCAB = 100000
EMBED = 768
BATCH = 4
SEQ = 2048

_INFO = plsc.get_sparse_core_info()
NC = _INFO.num_cores
NS = _INFO.num_subcores
NW = NC * NS
NFLAT = BATCH * SEQ
BPW = NFLAT // NW
CH = 64
NCHUNK = BPW // CH


def _make_kernel():
    mesh = plsc.VectorSubcoreMesh(core_axis_name="c", subcore_axis_name="s")

    @functools.partial(
        pl.kernel,
        mesh=mesh,
        out_type=jax.ShapeDtypeStruct((NFLAT, EMBED), jnp.float32),
        scratch_types=[
            pltpu.VMEM((BPW,), jnp.int32),
            pltpu.VMEM((CH, EMBED), jnp.float32),
            pltpu.VMEM((CH, EMBED), jnp.float32),
            pltpu.SemaphoreType.DMA,
        ],
    )
    def k(tok_hbm, idx_hbm, pos_hbm, out_hbm, idx_v, tok_v, pos_v, sem):
        wid = lax.axis_index("s") * NC + lax.axis_index("c")
        base = wid * BPW
        t0 = lax.rem(base, SEQ)
        pltpu.sync_copy(idx_hbm.at[pl.ds(base, BPW)], idx_v)
        for c in range(NCHUNK):
            tok_dma = pltpu.async_copy(
                tok_hbm.at[idx_v.at[pl.ds(c * CH, CH)]], tok_v, sem
            )
            pltpu.sync_copy(pos_hbm.at[pl.ds(t0 + c * CH, CH)], pos_v)
            tok_dma.wait()

            def add_row(r, _):
                for j in range(EMBED // 16):
                    sl = pl.ds(j * 16, 16)
                    tok_v[r, sl] = tok_v[r, sl] + pos_v[r, sl]
                return _

            lax.fori_loop(0, CH, add_row, 0)
            pltpu.sync_copy(tok_v, out_hbm.at[pl.ds(base + c * CH, CH)])

    return k


_k = _make_kernel()


def kernel(values, tok_table, pos_table):
    idx = values.reshape(-1).astype(jnp.int32)
    out = _k(tok_table, idx, pos_table)
    return out.reshape(BATCH, SEQ, EMBED)


# speedup vs baseline: 1.1900x; 1.1424x over previous
import functools

import jax
import jax.numpy as jnp
from jax import lax
from jax.experimental import pallas as pl
from jax.experimental.pallas import tpu as pltpu
from jax.experimental.pallas import tpu_sc as plsc

VOCAB = 100000
EMBED = 768
BATCH = 4
SEQ = 2048

_INFO = plsc.get_sparse_core_info()
NC = _INFO.num_cores
NS = _INFO.num_subcores
NW = NC * NS
NFLAT = BATCH * SEQ
BPW = NFLAT // NW
CH = 32
NCHUNK = BPW // CH


def _make_kernel():
    mesh = plsc.VectorSubcoreMesh(core_axis_name="c", subcore_axis_name="s")

    @functools.partial(
        pl.kernel,
        mesh=mesh,
        out_type=jax.ShapeDtypeStruct((NFLAT, EMBED), jnp.float32),
        scratch_types=[
            pltpu.VMEM((BPW,), jnp.int32),
            pltpu.VMEM((CH, EMBED), jnp.float32),
            pltpu.VMEM((CH, EMBED), jnp.float32),
            pltpu.VMEM((CH, EMBED), jnp.float32),
            pltpu.VMEM((CH, EMBED), jnp.float32),
            pltpu.SemaphoreType.DMA,
            pltpu.SemaphoreType.DMA,
            pltpu.SemaphoreType.DMA,
            pltpu.SemaphoreType.DMA,
            pltpu.SemaphoreType.DMA,
            pltpu.SemaphoreType.DMA,
        ],
    )
    def k(tok_hbm, idx_hbm, pos_hbm, out_hbm, idx_v,
          tok_a, tok_b, pos_a, pos_b,
          stok_a, stok_b, spos_a, spos_b, sout_a, sout_b):
        tok_bufs = (tok_a, tok_b)
        pos_bufs = (pos_a, pos_b)
        stoks = (stok_a, stok_b)
        sposs = (spos_a, spos_b)
        souts = (sout_a, sout_b)

        wid = lax.axis_index("s") * NC + lax.axis_index("c")
        base = wid * BPW
        t0 = lax.rem(base, SEQ)
        pltpu.sync_copy(idx_hbm.at[pl.ds(base, BPW)], idx_v)

        def start_in(c):
            i = c % 2
            g = pltpu.async_copy(
                tok_hbm.at[idx_v.at[pl.ds(c * CH, CH)]], tok_bufs[i], stoks[i]
            )
            p = pltpu.async_copy(
                pos_hbm.at[pl.ds(t0 + c * CH, CH)], pos_bufs[i], sposs[i]
            )
            return g, p

        in_flight = {0: start_in(0)}
        out_flight = {}
        for c in range(NCHUNK):
            i = c % 2
            if c + 1 < NCHUNK:
                if c - 1 in out_flight:
                    out_flight.pop(c - 1).wait()
                in_flight[c + 1] = start_in(c + 1)
            g, p = in_flight.pop(c)
            g.wait()
            p.wait()

            def add_row(r, _, _tok=tok_bufs[i], _pos=pos_bufs[i]):
                for j in range(EMBED // 16):
                    sl = pl.ds(j * 16, 16)
                    plsc.addupdate(_tok.at[r, sl], _pos[r, sl])
                return _

            lax.fori_loop(0, CH, add_row, 0)
            out_flight[c] = pltpu.async_copy(
                tok_bufs[i], out_hbm.at[pl.ds(base + c * CH, CH)], souts[i]
            )
        for c in sorted(out_flight):
            out_flight.pop(c).wait()

    return k


_k = _make_kernel()


def kernel(values, tok_table, pos_table):
    idx = values.reshape(-1).astype(jnp.int32)
    out = _k(tok_table, idx, pos_table)
    return out.reshape(BATCH, SEQ, EMBED)


# speedup vs baseline: 1.2664x; 1.0642x over previous
import functools

import jax
import jax.numpy as jnp
from jax import lax
from jax.experimental import pallas as pl
from jax.experimental.pallas import tpu as pltpu
from jax.experimental.pallas import tpu_sc as plsc

VOCAB = 100000
EMBED = 768
BATCH = 4
SEQ = 2048

_INFO = plsc.get_sparse_core_info()
NC = _INFO.num_cores
NS = _INFO.num_subcores
NW = NC * NS
NFLAT = BATCH * SEQ
TPW = SEQ // NW
TCH = 8
NCHUNK = TPW // TCH
CROWS = BATCH * TCH
BPW = BATCH * TPW
NLANE = 16
NVEC = EMBED // NLANE


def _make_kernel():
    mesh = plsc.VectorSubcoreMesh(core_axis_name="c", subcore_axis_name="s")

    @functools.partial(
        pl.kernel,
        mesh=mesh,
        out_type=jax.ShapeDtypeStruct((NFLAT, EMBED), jnp.float32),
        scratch_types=[
            pltpu.VMEM((BPW,), jnp.int32),
            pltpu.VMEM((CROWS, EMBED), jnp.float32),
            pltpu.VMEM((CROWS, EMBED), jnp.float32),
            pltpu.VMEM((CROWS, EMBED), jnp.float32),
            pltpu.VMEM((CROWS, EMBED), jnp.float32),
            pltpu.VMEM((TCH, EMBED), jnp.float32),
            pltpu.VMEM((TCH, EMBED), jnp.float32),
            pltpu.SemaphoreType.DMA,
            pltpu.SemaphoreType.DMA,
            pltpu.SemaphoreType.DMA,
            pltpu.SemaphoreType.DMA,
            pltpu.SemaphoreType.DMA,
            pltpu.SemaphoreType.DMA,
            pltpu.SemaphoreType.DMA,
            pltpu.SemaphoreType.DMA,
            pltpu.SemaphoreType.DMA,
            pltpu.SemaphoreType.DMA,
        ],
    )
    def k(tok_hbm, idx_hbm, pos_hbm, out_hbm, idx_v,
          tok_a, tok_b, tok_c, tok_d, pos_a, pos_b,
          stok_a, stok_b, stok_c, stok_d, spos_a, spos_b,
          sout_a, sout_b, sout_c, sout_d):
        tok_bufs = (tok_a, tok_b, tok_c, tok_d)
        pos_bufs = (pos_a, pos_b)
        stoks = (stok_a, stok_b, stok_c, stok_d)
        sposs = (spos_a, spos_b)
        souts = (sout_a, sout_b, sout_c, sout_d)

        wid = lax.axis_index("s") * NC + lax.axis_index("c")
        ibase = wid * BPW
        t0 = wid * TPW
        pltpu.sync_copy(idx_hbm.at[pl.ds(ibase, BPW)], idx_v)

        def start_gather(c):
            return pltpu.async_copy(
                tok_hbm.at[idx_v.at[pl.ds(c * CROWS, CROWS)]],
                tok_bufs[c % 4], stoks[c % 4],
            )

        def start_pos(c):
            return pltpu.async_copy(
                pos_hbm.at[pl.ds(t0 + c * TCH, TCH)],
                pos_bufs[c % 2], sposs[c % 2],
            )

        gathers = {0: start_gather(0), 1: start_gather(1)}
        poss = {0: start_pos(0)}
        outs = {}
        for c in range(NCHUNK):
            gathers.pop(c).wait()
            poss.pop(c).wait()
            tok_v = tok_bufs[c % 4]
            pos_v = pos_bufs[c % 2]

            def add_pos(t, _, _tok=tok_v, _pos=pos_v):
                for j in range(NVEC):
                    sl = pl.ds(j * NLANE, NLANE)
                    pv = _pos[t, sl]
                    for b in range(BATCH):
                        plsc.addupdate(_tok.at[b * TCH + t, sl], pv)
                return _

            lax.fori_loop(0, TCH, add_pos, 0)

            outs[c] = [
                pltpu.async_copy(
                    tok_v.at[pl.ds(b * TCH, TCH)],
                    out_hbm.at[pl.ds(b * SEQ + t0 + c * TCH, TCH)],
                    souts[c % 4],
                )
                for b in range(BATCH)
            ]
            if c - 2 in outs:
                for d in outs.pop(c - 2):
                    d.wait()
            if c + 2 < NCHUNK:
                gathers[c + 2] = start_gather(c + 2)
            if c + 1 < NCHUNK:
                poss[c + 1] = start_pos(c + 1)
        for c in sorted(outs):
            for d in outs.pop(c):
                d.wait()

    return k


_k = _make_kernel()


def kernel(values, tok_table, pos_table):
    idx = (
        values.astype(jnp.int32)
        .reshape(BATCH, NW, NCHUNK, TCH)
        .transpose(1, 2, 0, 3)
        .reshape(-1)
    )
    out = _k(tok_table, idx, pos_table)
    return out.reshape(BATCH, SEQ, EMBED)
